# baseline (speedup 1.0000x reference)
import functools

import jax
import jax.numpy as jnp
from jax.experimental import pallas as pl
from jax.experimental.pallas import tpu as pltpu

_NEG_SLOPE = 0.01
_BM = 2048
_BK = 2560
_BK1 = 1792

_NT = (((1,), (1,)), ((), ()))


def _leaky(t):
    return jnp.where(t >= 0, t, _NEG_SLOPE * t)


def _mask_cols(blk, valid):
    it = jax.lax.broadcasted_iota(jnp.int32, blk.shape, 1)
    return jnp.where(it < valid, blk, 0)


def _mmt_kernel(x_ref, w_ref, o_ref):
    o_ref[...] = jax.lax.dot_general(
        w_ref[...], x_ref[...], (((0,), (1,)), ((), ())),
        preferred_element_type=jnp.float32).astype(jnp.bfloat16)


def _small_mmt(x, w, bm):
    n, d = x.shape
    fo = w.shape[1]
    return pl.pallas_call(
        _mmt_kernel,
        grid=(pl.cdiv(n, bm),),
        in_specs=[pl.BlockSpec((bm, d), lambda i: (i, 0)),
                  pl.BlockSpec((d, fo), lambda i: (0, 0))],
        out_specs=pl.BlockSpec((fo, bm), lambda i: (0, i)),
        out_shape=jax.ShapeDtypeStruct((fo, n), jnp.bfloat16),
    )(x, w)


def _mask_rows(blk, valid):
    it = jax.lax.broadcasted_iota(jnp.int32, blk.shape, 0)
    return jnp.where(it < valid, blk, 0)


def _acc_tn(bt_ref, at_bf, acc_ref, k, nk, rem, bk):
    def _step(mask):
        bt = bt_ref[...]
        a = at_bf
        if mask:
            bt = _mask_cols(bt, rem)
            a = _mask_rows(a, rem)
        acc_ref[...] += jnp.dot(bt, a, preferred_element_type=jnp.float32)

    if rem == bk:
        _step(False)
    else:
        @pl.when(k < nk - 1)
        def _():
            _step(False)

        @pl.when(k == nk - 1)
        def _():
            _step(True)


def _first_kernel(adj_ref, bt_ref, wt_ref, o_ref, adjbt_ref, acc_ref,
                  *, nk, rem):
    k = pl.program_id(1)

    @pl.when(k == 0)
    def _():
        acc_ref[...] = jnp.zeros_like(acc_ref)

    abt = adj_ref[...].astype(jnp.bfloat16).T
    adjbt_ref[...] = abt[None]
    _acc_tn(bt_ref, abt, acc_ref, k, nk, rem, _BK1)

    @pl.when(k == nk - 1)
    def _():
        h = _leaky(acc_ref[...])
        o_ref[...] = jnp.dot(
            wt_ref[...], h,
            preferred_element_type=jnp.float32).astype(jnp.bfloat16)


def _gcn_first(adj, bt, wt_next):
    n = adj.shape[0]
    wd = bt.shape[0]
    wo = wt_next.shape[0]
    nk = pl.cdiv(n, _BK1)
    rem = n - (nk - 1) * _BK1
    return pl.pallas_call(
        functools.partial(_first_kernel, nk=nk, rem=rem),
        grid=(pl.cdiv(n, _BM), nk),
        in_specs=[pl.BlockSpec((_BM, _BK1), lambda j, k: (j, k)),
                  pl.BlockSpec((wd, _BK1), lambda j, k: (0, k)),
                  pl.BlockSpec((wo, wd), lambda j, k: (0, 0))],
        out_specs=(pl.BlockSpec((wo, _BM), lambda j, k: (0, j)),
                   pl.BlockSpec((1, _BK1, _BM), lambda j, k: (j, k, 0))),
        out_shape=(jax.ShapeDtypeStruct((wo, n), jnp.bfloat16),
                   jax.ShapeDtypeStruct((pl.cdiv(n, _BM), n, _BM),
                                        jnp.bfloat16)),
        scratch_shapes=[pltpu.VMEM((wd, _BM), jnp.float32)],
        compiler_params=pltpu.CompilerParams(
            dimension_semantics=("parallel", "arbitrary")),
    )(adj, bt, wt_next)


def _acc_tn_val(bt, at_bf, acc_ref, sl, k, nk, rem, bk):
    def _step(mask):
        b, a = bt, at_bf
        if mask:
            b = _mask_cols(b, rem)
            a = _mask_rows(a, rem)
        acc_ref[sl, :] += jnp.dot(b, a, preferred_element_type=jnp.float32)

    if rem == bk:
        _step(False)
    else:
        @pl.when(k < nk - 1)
        def _():
            _step(False)

        @pl.when(k == nk - 1)
        def _():
            _step(True)


def _fused_kernel(adj_ref, bt_ref, w3t_ref, w4ct_ref, fcw_ref, fcb_ref,
                  mu_ref, lv_ref, xr_ref, zb_ref,
                  acc_ref, s1_ref, s2_ref, *, nk, rem, wds):
    p = pl.program_id(0)
    j = pl.program_id(1)
    k = pl.program_id(2)
    w1, w2, w3 = wds
    bm = acc_ref.shape[1]
    bk = adj_ref.shape[1]

    @pl.when(k == 0)
    def _():
        acc_ref[...] = jnp.zeros_like(acc_ref)

    at = adj_ref[0]

    @pl.when(p == 0)
    def _():
        _acc_tn_val(bt_ref[...], at, acc_ref, slice(0, w1), k, nk, rem, bk)

        @pl.when(k == nk - 1)
        def _():
            h = _leaky(acc_ref[0:w1, :])
            s1_ref[:, pl.ds(j * bm, bm)] = jnp.dot(
                w3t_ref[...], h,
                preferred_element_type=jnp.float32).astype(jnp.bfloat16)

    @pl.when(p == 1)
    def _():
        _acc_tn_val(s1_ref[:, pl.ds(k * bk, bk)], at, acc_ref,
                    slice(0, w2), k, nk, rem, bk)

        @pl.when(k == nk - 1)
        def _():
            h = _leaky(acc_ref[0:w2, :])
            s2_ref[:, pl.ds(j * bm, bm)] = jnp.dot(
                w4ct_ref[...], h,
                preferred_element_type=jnp.float32).astype(jnp.bfloat16)

    @pl.when(p == 2)
    def _():
        _acc_tn_val(s2_ref[:, pl.ds(k * bk, bk)], at, acc_ref,
                    slice(0, w3), k, nk, rem, bk)

        @pl.when(k == nk - 1)
        def _():
            h4 = w3 // 2
            h = _leaky(acc_ref[0:w3, :]).T
            mu = h[:, :h4]
            mu_ref[...] = mu
            lv_ref[...] = h[:, h4:]
            zb_ref[...] = mu.astype(jnp.bfloat16)
            xr_ref[...] = jnp.dot(
                mu, fcw_ref[...],
                preferred_element_type=jnp.float32) + fcb_ref[...]


def _gcn_fused(adj_bf, bt, w3t, w4ct, fcw, fcb_row):
    nj, n, bm = adj_bf.shape
    w1 = bt.shape[0]
    w2 = w3t.shape[0]
    w3 = w4ct.shape[0]
    h4 = w3 // 2
    d = fcw.shape[1]
    nk = pl.cdiv(n, _BK)
    rem = n - (nk - 1) * _BK
    npad = nj * bm

    def _out_idx(p, j, k):
        return (jnp.where(p == 2, j, 0), 0)

    out_shapes = (jax.ShapeDtypeStruct((n, h4), jnp.float32),
                  jax.ShapeDtypeStruct((n, h4), jnp.float32),
                  jax.ShapeDtypeStruct((n, d), jnp.float32),
                  jax.ShapeDtypeStruct((n, h4), jnp.bfloat16))
    return pl.pallas_call(
        functools.partial(_fused_kernel, nk=nk, rem=rem, wds=(w1, w2, w3)),
        grid=(3, nj, nk),
        in_specs=[pl.BlockSpec((1, _BK, bm), lambda p, j, k: (j, k, 0)),
                  pl.BlockSpec((w1, _BK), lambda p, j, k: (0, k)),
                  pl.BlockSpec((w2, w1), lambda p, j, k: (0, 0)),
                  pl.BlockSpec((w3, w2), lambda p, j, k: (0, 0)),
                  pl.BlockSpec((h4, d), lambda p, j, k: (0, 0)),
                  pl.BlockSpec((1, d), lambda p, j, k: (0, 0))],
        out_specs=(pl.BlockSpec((bm, h4), _out_idx),
                   pl.BlockSpec((bm, h4), _out_idx),
                   pl.BlockSpec((bm, d), _out_idx),
                   pl.BlockSpec((bm, h4), _out_idx)),
        out_shape=out_shapes,
        scratch_shapes=[pltpu.VMEM((w3, bm), jnp.float32),
                        pltpu.VMEM((w2, npad), jnp.bfloat16),
                        pltpu.VMEM((w3, npad), jnp.bfloat16)],
        compiler_params=pltpu.CompilerParams(
            dimension_semantics=("arbitrary", "arbitrary", "arbitrary")),
    )(adj_bf, bt, w3t, w4ct, fcw, fcb_row)


def _dc_kernel(zi_ref, zj_ref, o_ref):
    o_ref[...] = jax.lax.dot_general(
        zi_ref[...], zj_ref[...], _NT, preferred_element_type=jnp.float32)


def _decode(z):
    n, h = z.shape
    nb = pl.cdiv(n, _BM)
    return pl.pallas_call(
        _dc_kernel,
        grid=(nb, nb),
        in_specs=[pl.BlockSpec((_BM, h), lambda i, j: (i, 0)),
                  pl.BlockSpec((_BM, h), lambda i, j: (j, 0))],
        out_specs=pl.BlockSpec((_BM, _BM), lambda i, j: (i, j)),
        out_shape=jax.ShapeDtypeStruct((n, n), jnp.float32),
        compiler_params=pltpu.CompilerParams(
            dimension_semantics=("parallel", "parallel")),
    )(z, z)


def kernel(x, adj, W1, W2, W3, W4, W4s, fcW, fcb,
           bn_gamma, bn_beta, bn_mean, bn_var):
    scale = bn_gamma / jnp.sqrt(bn_var + 1e-5)
    fcWp = fcW * scale[None, :]
    fcbp = (((fcb - bn_mean) * scale + bn_beta))[None, :]
    W2t = W2.T
    W3t = W3.T
    W4ct = jnp.concatenate([W4, W4s], axis=1).T

    xW1t = _small_mmt(x, W1, _BM)
    h1W2t, adj_bf = _gcn_first(adj, xW1t, W2t)
    mu, lv, xr, zb = _gcn_fused(adj_bf, h1W2t, W3t, W4ct, fcWp, fcbp)
    dc = _decode(zb)
    return (dc, mu, lv, mu, xr)
    h2W3t = _gcn_layer(adj_bf, h1W2t, W3t)
    h3W4t = _gcn_layer(adj_bf, h2W3t, W4ct)
    mu, lv, xr, zb = _gcn_final(adj_bf, h3W4t, fcWp, fcbp)
    dc = _decode(zb)
    return (dc, mu, lv, mu, xr)


# speedup vs baseline: 1.0105x; 1.0105x over previous
import functools

import jax
import jax.numpy as jnp
from jax.experimental import pallas as pl
from jax.experimental.pallas import tpu as pltpu

_NEG_SLOPE = 0.01
_BM = 2048
_BK = 2560
_BK1 = 1536

_NT = (((1,), (1,)), ((), ()))


def _leaky(t):
    return jnp.where(t >= 0, t, _NEG_SLOPE * t)


def _mask_cols(blk, valid):
    it = jax.lax.broadcasted_iota(jnp.int32, blk.shape, 1)
    return jnp.where(it < valid, blk, 0)


def _mmt_kernel(x_ref, w_ref, o_ref):
    o_ref[...] = jax.lax.dot_general(
        w_ref[...], x_ref[...], (((0,), (1,)), ((), ())),
        preferred_element_type=jnp.float32).astype(jnp.bfloat16)


def _small_mmt(x, w, bm):
    n, d = x.shape
    fo = w.shape[1]
    return pl.pallas_call(
        _mmt_kernel,
        grid=(pl.cdiv(n, bm),),
        in_specs=[pl.BlockSpec((bm, d), lambda i: (i, 0)),
                  pl.BlockSpec((d, fo), lambda i: (0, 0))],
        out_specs=pl.BlockSpec((fo, bm), lambda i: (0, i)),
        out_shape=jax.ShapeDtypeStruct((fo, n), jnp.bfloat16),
    )(x, w)


def _mask_rows(blk, valid):
    it = jax.lax.broadcasted_iota(jnp.int32, blk.shape, 0)
    return jnp.where(it < valid, blk, 0)


def _acc_tn(bt_ref, at_bf, acc_ref, k, nk, rem, bk):
    def _step(mask):
        bt = bt_ref[...]
        a = at_bf
        if mask:
            bt = _mask_cols(bt, rem)
            a = _mask_rows(a, rem)
        acc_ref[...] += jnp.dot(bt, a, preferred_element_type=jnp.float32)

    if rem == bk:
        _step(False)
    else:
        @pl.when(k < nk - 1)
        def _():
            _step(False)

        @pl.when(k == nk - 1)
        def _():
            _step(True)


def _first_kernel(adj_ref, bt_ref, wt_ref, o_ref, adjbt_ref, acc_ref,
                  *, nk, rem):
    k = pl.program_id(1)

    @pl.when(k == 0)
    def _():
        acc_ref[...] = jnp.zeros_like(acc_ref)

    abt = adj_ref[...].astype(jnp.bfloat16).T
    adjbt_ref[...] = abt[None]
    _acc_tn(bt_ref, abt, acc_ref, k, nk, rem, _BK1)

    @pl.when(k == nk - 1)
    def _():
        h = _leaky(acc_ref[...])
        o_ref[...] = jnp.dot(
            wt_ref[...], h,
            preferred_element_type=jnp.float32).astype(jnp.bfloat16)


def _gcn_first(adj, bt, wt_next):
    n = adj.shape[0]
    wd = bt.shape[0]
    wo = wt_next.shape[0]
    nk = pl.cdiv(n, _BK1)
    rem = n - (nk - 1) * _BK1
    return pl.pallas_call(
        functools.partial(_first_kernel, nk=nk, rem=rem),
        grid=(pl.cdiv(n, _BM), nk),
        in_specs=[pl.BlockSpec((_BM, _BK1), lambda j, k: (j, k)),
                  pl.BlockSpec((wd, _BK1), lambda j, k: (0, k)),
                  pl.BlockSpec((wo, wd), lambda j, k: (0, 0))],
        out_specs=(pl.BlockSpec((wo, _BM), lambda j, k: (0, j)),
                   pl.BlockSpec((1, _BK1, _BM), lambda j, k: (j, k, 0))),
        out_shape=(jax.ShapeDtypeStruct((wo, n), jnp.bfloat16),
                   jax.ShapeDtypeStruct((pl.cdiv(n, _BM), n, _BM),
                                        jnp.bfloat16)),
        scratch_shapes=[pltpu.VMEM((wd, _BM), jnp.float32)],
        compiler_params=pltpu.CompilerParams(
            dimension_semantics=("parallel", "arbitrary")),
    )(adj, bt, wt_next)


def _acc_tn_val(bt, at_bf, acc_ref, sl, k, nk, rem, bk):
    def _step(mask):
        b, a = bt, at_bf
        if mask:
            b = _mask_cols(b, rem)
            a = _mask_rows(a, rem)
        acc_ref[sl, :] += jnp.dot(b, a, preferred_element_type=jnp.float32)

    if rem == bk:
        _step(False)
    else:
        @pl.when(k < nk - 1)
        def _():
            _step(False)

        @pl.when(k == nk - 1)
        def _():
            _step(True)


def _fused_kernel(adj_ref, bt_ref, w3t_ref, w4ct_ref, fcw_ref, fcb_ref,
                  mu_ref, lv_ref, xr_ref, zb_ref,
                  acc_ref, s1_ref, s2_ref, *, nk, rem, wds):
    p = pl.program_id(0)
    j = pl.program_id(1)
    k = pl.program_id(2)
    w1, w2, w3 = wds
    bm = acc_ref.shape[1]
    bk = adj_ref.shape[1]

    @pl.when(k == 0)
    def _():
        acc_ref[...] = jnp.zeros_like(acc_ref)

    at = adj_ref[0]

    @pl.when(p == 0)
    def _():
        _acc_tn_val(bt_ref[...], at, acc_ref, slice(0, w1), k, nk, rem, bk)

        @pl.when(k == nk - 1)
        def _():
            h = _leaky(acc_ref[0:w1, :])
            s1_ref[:, pl.ds(j * bm, bm)] = jnp.dot(
                w3t_ref[...], h,
                preferred_element_type=jnp.float32).astype(jnp.bfloat16)

    @pl.when(p == 1)
    def _():
        _acc_tn_val(s1_ref[:, pl.ds(k * bk, bk)], at, acc_ref,
                    slice(0, w2), k, nk, rem, bk)

        @pl.when(k == nk - 1)
        def _():
            h = _leaky(acc_ref[0:w2, :])
            s2_ref[:, pl.ds(j * bm, bm)] = jnp.dot(
                w4ct_ref[...], h,
                preferred_element_type=jnp.float32).astype(jnp.bfloat16)

    @pl.when(p == 2)
    def _():
        _acc_tn_val(s2_ref[:, pl.ds(k * bk, bk)], at, acc_ref,
                    slice(0, w3), k, nk, rem, bk)

        @pl.when(k == nk - 1)
        def _():
            h4 = w3 // 2
            h = _leaky(acc_ref[0:w3, :]).T
            mu = h[:, :h4]
            mu_ref[...] = mu
            lv_ref[...] = h[:, h4:]
            zb_ref[...] = mu.astype(jnp.bfloat16)
            xr_ref[...] = jnp.dot(
                mu, fcw_ref[...],
                preferred_element_type=jnp.float32) + fcb_ref[...]


def _gcn_fused(adj_bf, bt, w3t, w4ct, fcw, fcb_row):
    nj, n, bm = adj_bf.shape
    w1 = bt.shape[0]
    w2 = w3t.shape[0]
    w3 = w4ct.shape[0]
    h4 = w3 // 2
    d = fcw.shape[1]
    nk = pl.cdiv(n, _BK)
    rem = n - (nk - 1) * _BK
    npad = nj * bm

    def _out_idx(p, j, k):
        return (jnp.where(p == 2, j, 0), 0)

    out_shapes = (jax.ShapeDtypeStruct((n, h4), jnp.float32),
                  jax.ShapeDtypeStruct((n, h4), jnp.float32),
                  jax.ShapeDtypeStruct((n, d), jnp.float32),
                  jax.ShapeDtypeStruct((n, h4), jnp.bfloat16))
    return pl.pallas_call(
        functools.partial(_fused_kernel, nk=nk, rem=rem, wds=(w1, w2, w3)),
        grid=(3, nj, nk),
        in_specs=[pl.BlockSpec((1, _BK, bm), lambda p, j, k: (j, k, 0)),
                  pl.BlockSpec((w1, _BK), lambda p, j, k: (0, k)),
                  pl.BlockSpec((w2, w1), lambda p, j, k: (0, 0)),
                  pl.BlockSpec((w3, w2), lambda p, j, k: (0, 0)),
                  pl.BlockSpec((h4, d), lambda p, j, k: (0, 0)),
                  pl.BlockSpec((1, d), lambda p, j, k: (0, 0))],
        out_specs=(pl.BlockSpec((bm, h4), _out_idx),
                   pl.BlockSpec((bm, h4), _out_idx),
                   pl.BlockSpec((bm, d), _out_idx),
                   pl.BlockSpec((bm, h4), _out_idx)),
        out_shape=out_shapes,
        scratch_shapes=[pltpu.VMEM((w3, bm), jnp.float32),
                        pltpu.VMEM((w2, npad), jnp.bfloat16),
                        pltpu.VMEM((w3, npad), jnp.bfloat16)],
        compiler_params=pltpu.CompilerParams(
            dimension_semantics=("arbitrary", "arbitrary", "arbitrary")),
    )(adj_bf, bt, w3t, w4ct, fcw, fcb_row)


def _dc_kernel(zi_ref, zj_ref, o_ref):
    o_ref[...] = jax.lax.dot_general(
        zi_ref[...], zj_ref[...], _NT, preferred_element_type=jnp.float32)


def _decode(z):
    n, h = z.shape
    nb = pl.cdiv(n, _BM)
    return pl.pallas_call(
        _dc_kernel,
        grid=(nb, nb),
        in_specs=[pl.BlockSpec((_BM, h), lambda i, j: (i, 0)),
                  pl.BlockSpec((_BM, h), lambda i, j: (j, 0))],
        out_specs=pl.BlockSpec((_BM, _BM), lambda i, j: (i, j)),
        out_shape=jax.ShapeDtypeStruct((n, n), jnp.float32),
        compiler_params=pltpu.CompilerParams(
            dimension_semantics=("parallel", "parallel")),
    )(z, z)


def kernel(x, adj, W1, W2, W3, W4, W4s, fcW, fcb,
           bn_gamma, bn_beta, bn_mean, bn_var):
    scale = bn_gamma / jnp.sqrt(bn_var + 1e-5)
    fcWp = fcW * scale[None, :]
    fcbp = (((fcb - bn_mean) * scale + bn_beta))[None, :]
    W2t = W2.T
    W3t = W3.T
    W4ct = jnp.concatenate([W4, W4s], axis=1).T

    xW1t = _small_mmt(x, W1, _BM)
    h1W2t, adj_bf = _gcn_first(adj, xW1t, W2t)
    mu, lv, xr, zb = _gcn_fused(adj_bf, h1W2t, W3t, W4ct, fcWp, fcbp)
    dc = _decode(zb)
    return (dc, mu, lv, mu, xr)
    h2W3t = _gcn_layer(adj_bf, h1W2t, W3t)
    h3W4t = _gcn_layer(adj_bf, h2W3t, W4ct)
    mu, lv, xr, zb = _gcn_final(adj_bf, h3W4t, fcWp, fcbp)
    dc = _decode(zb)
    return (dc, mu, lv, mu, xr)
